# baseline (speedup 1.0000x reference)
import functools

import jax
import jax.numpy as jnp
from jax import lax
from jax.experimental import pallas as pl
from jax.experimental.pallas import tpu as pltpu
from jax.experimental.pallas import tpu_sc as plsc

NUM_EMBEDDINGS = 1000000
FEATURES = 64
BATCH = 4096
SEQ = 20
TOTAL = BATCH * SEQ
CLIP = 127.0
MAGIC = 1.5 * (2.0 ** 23)
CHUNK = 128


def _perm(x, idx):
    return lax.gather(
        x, idx[:, None],
        lax.GatherDimensionNumbers(
            offset_dims=(), collapsed_slice_dims=(0,), start_index_map=(0,)),
        (1,), mode=lax.GatherScatterMode.PROMISE_IN_BOUNDS)


def _dequant_rows(rows_v, r, perms):
    a = [rows_v[r, pl.ds(16 * k, 16)] for k in range(FEATURES // 16)]
    m = jnp.maximum(jnp.maximum(jnp.abs(a[0]), jnp.abs(a[1])),
                    jnp.maximum(jnp.abs(a[2]), jnp.abs(a[3])))
    for p in perms:
        m = jnp.maximum(m, _perm(m, p))
    mvec = jnp.maximum(m, 1e-9)
    scale = CLIP / mvec
    inv = mvec * (1.0 / CLIP)
    for k in range(FEATURES // 16):
        t = a[k] * scale
        t = jnp.minimum(jnp.maximum(t, -CLIP), CLIP)
        q = (t + MAGIC) - MAGIC
        rows_v[r, pl.ds(16 * k, 16)] = q * inv


def _sc_body(nc, chunks, table_hbm, idx_hbm, out_hbm, idx_v, rows_v, sem):
    wid = lax.axis_index("s") * nc + lax.axis_index("c")
    pltpu.sync_copy(idx_hbm.at[wid], idx_v)
    lanes = lax.iota(jnp.int32, 16)
    perms = [lanes ^ b for b in (8, 4, 2, 1)]

    def chunk_step(j, carry):
        pltpu.async_copy(table_hbm.at[idx_v.at[j]], rows_v, sem).wait()

        def row_step(r, c2):
            _dequant_rows(rows_v, r, perms)
            return c2

        lax.fori_loop(0, CHUNK, row_step, 0, unroll=2)
        pltpu.sync_copy(
            rows_v, out_hbm.at[pl.ds((wid * chunks + j) * CHUNK, CHUNK)])
        return carry

    lax.fori_loop(0, chunks, chunk_step, 0)


def kernel(inputs, embedding):
    info = plsc.get_sparse_core_info()
    nc, ns = info.num_cores, info.num_subcores
    nw = nc * ns
    chunks = TOTAL // (nw * CHUNK)
    idx = inputs.reshape(nw, chunks, CHUNK)

    mesh = plsc.VectorSubcoreMesh(core_axis_name="c", subcore_axis_name="s")
    k = pl.kernel(
        functools.partial(_sc_body, nc, chunks),
        mesh=mesh,
        out_type=jax.ShapeDtypeStruct((TOTAL, FEATURES), jnp.float32),
        scratch_types=[
            pltpu.VMEM((chunks, CHUNK), jnp.int32),
            pltpu.VMEM((CHUNK, FEATURES), jnp.float32),
            pltpu.SemaphoreType.DMA,
        ],
        compiler_params=pltpu.CompilerParams(use_tc_tiling_on_sc=False),
    )
    out = k(embedding, idx)
    return out.reshape(BATCH, SEQ, FEATURES)
